# Initial kernel scaffold; baseline (speedup 1.0000x reference)
import jax
import jax.numpy as jnp
from jax.experimental import pallas as pl


def kernel(pieces_ids, color_ids, position_emb, piece_emb, color_emb):
    raise NotImplementedError("write your pallas kernel here")


# speedup vs baseline: 12.2155x; 12.2155x over previous
import jax
import jax.numpy as jnp
from jax.experimental import pallas as pl

SEQ = 64
EMBED = 64
NJ = 21
KPAD = 32


def _tc_body(p_ref, c_ref, pos_ref, piece_ref, color_ref, out_ref):
    bb = p_ref.shape[0]
    j = p_ref[...] * 3 + c_ref[...]
    j3 = j[:, :, None]
    iota = jax.lax.broadcasted_iota(jnp.int32, (bb, SEQ, KPAD), 2)
    oh = (j3 == iota).astype(jnp.float32).reshape(bb * SEQ, KPAD)
    piece3 = jnp.broadcast_to(piece_ref[...][:, None, :], (7, 3, EMBED)).reshape(NJ, EMBED)
    color7 = jnp.broadcast_to(color_ref[...][None, :, :], (7, 3, EMBED)).reshape(NJ, EMBED)
    joint = piece3 + color7
    joint_pad = jnp.concatenate(
        [joint, jnp.zeros((KPAD - NJ, EMBED), jnp.float32)], axis=0)
    acc = jnp.dot(oh, joint_pad, preferred_element_type=jnp.float32)
    out_ref[...] = acc.reshape(bb, SEQ, EMBED) + pos_ref[...][None, :, :]


def kernel(pieces_ids, color_ids, position_emb, piece_emb, color_emb):
    B = pieces_ids.shape[0]
    Bb = 128
    p32 = pieces_ids.astype(jnp.int32)
    c32 = color_ids.astype(jnp.int32)
    out = pl.pallas_call(
        _tc_body,
        grid=(B // Bb,),
        in_specs=[
            pl.BlockSpec((Bb, SEQ), lambda i: (i, 0)),
            pl.BlockSpec((Bb, SEQ), lambda i: (i, 0)),
            pl.BlockSpec((SEQ, EMBED), lambda i: (0, 0)),
            pl.BlockSpec((7, EMBED), lambda i: (0, 0)),
            pl.BlockSpec((3, EMBED), lambda i: (0, 0)),
        ],
        out_specs=pl.BlockSpec((Bb, SEQ, EMBED), lambda i: (i, 0, 0)),
        out_shape=jax.ShapeDtypeStruct((B, SEQ, EMBED), jnp.float32),
    )(p32, c32, position_emb, piece_emb, color_emb)
    return out
